# Initial kernel scaffold; baseline (speedup 1.0000x reference)
import functools

import jax
import jax.numpy as jnp
from jax import lax
from jax.experimental import pallas as pl
from jax.experimental.pallas import tpu as pltpu
from jax.experimental.pallas import tpu_sc as plsc

N = 10000
F = 128
E = 320000
B = 2
NSUB = 16
NCORE = 2
CHUNK = 128
EPS = 20096
E_PAD = EPS * NSUB
NCHUNK = EPS // CHUNK
GARB = N
ACC_ROWS = N + 8
RPS = N // NSUB

_MESH = plsc.VectorSubcoreMesh(core_axis_name="c", subcore_axis_name="s")


def _make_agg(D):
    @functools.partial(
        pl.kernel,
        out_type=jax.ShapeDtypeStruct((NCORE * N, D), jnp.float32),
        mesh=_MESH,
        scratch_types=[
            pltpu.VMEM((CHUNK,), jnp.int32),
            pltpu.VMEM((CHUNK,), jnp.int32),
            pltpu.VMEM((CHUNK,), jnp.int32),
            pltpu.VMEM((CHUNK,), jnp.int32),
            pltpu.VMEM((CHUNK, D), jnp.float32),
            pltpu.VMEM_SHARED((ACC_ROWS, D), jnp.float32),
            pltpu.SemaphoreType.DMA,
        ],
    )
    def agg(z_hbm, src_hbm, dst_hbm, out_hbm, sbuf, dbuf, gbuf, wbuf, rows, acc, sem):
        core = lax.axis_index("c")
        sub = lax.axis_index("s")
        r0 = sub * RPS
        pltpu.sync_copy(z_hbm.at[pl.ds(core * N + r0, RPS)], acc.at[pl.ds(r0, RPS)])
        plsc.subcore_barrier()
        base = sub * EPS

        def body(ci, carry):
            off = base + ci * CHUNK
            pltpu.sync_copy(src_hbm.at[pl.ds(off, CHUNK)], sbuf)
            pltpu.sync_copy(dst_hbm.at[pl.ds(off, CHUNK)], dbuf)
            for j in range(CHUNK // 16):
                sl = pl.ds(j * 16, 16)
                s16 = sbuf[sl]
                d16 = dbuf[sl]
                gbuf[sl] = s16 + core * N
                wbuf[sl] = jnp.where(s16 == d16, GARB, d16)
            pltpu.async_copy(z_hbm.at[gbuf], rows, sem).wait()
            pltpu.sync_copy(rows, acc.at[wbuf], add=True)
            return carry

        lax.fori_loop(0, NCHUNK, body, 0)
        plsc.subcore_barrier()
        pltpu.sync_copy(acc.at[pl.ds(r0, RPS)], out_hbm.at[pl.ds(core * N + r0, RPS)])

    return agg


_agg128 = _make_agg(F)
_agg16 = _make_agg(16)


@functools.partial(
    pl.kernel,
    out_type=jax.ShapeDtypeStruct((NCORE * N, 16), jnp.float32),
    mesh=_MESH,
    scratch_types=[
        pltpu.VMEM((CHUNK,), jnp.int32),
        pltpu.VMEM((CHUNK,), jnp.int32),
        pltpu.VMEM((CHUNK,), jnp.int32),
        pltpu.VMEM((CHUNK, 16), jnp.float32),
        pltpu.VMEM_SHARED((ACC_ROWS, 16), jnp.float32),
    ],
)
def _degrees(src_hbm, dst_hbm, init_hbm, out_hbm, sbuf, dbuf, wbuf, ones, acc):
    core = lax.axis_index("c")
    sub = lax.axis_index("s")
    r0 = sub * RPS
    pltpu.sync_copy(init_hbm.at[pl.ds(0, CHUNK)], ones)
    pltpu.sync_copy(init_hbm.at[pl.ds(r0, RPS)], acc.at[pl.ds(r0, RPS)])
    plsc.subcore_barrier()
    base = sub * EPS

    def body(ci, carry):
        off = base + ci * CHUNK
        pltpu.sync_copy(src_hbm.at[pl.ds(off, CHUNK)], sbuf)
        pltpu.sync_copy(dst_hbm.at[pl.ds(off, CHUNK)], dbuf)
        for j in range(CHUNK // 16):
            sl = pl.ds(j * 16, 16)
            s16 = sbuf[sl]
            d16 = dbuf[sl]
            i16 = jnp.where(core == 0, s16, d16)
            wbuf[sl] = jnp.where(s16 == d16, GARB, i16)
        pltpu.sync_copy(ones, acc.at[wbuf], add=True)
        return carry

    lax.fori_loop(0, NCHUNK, body, 0)
    plsc.subcore_barrier()
    pltpu.sync_copy(acc.at[pl.ds(r0, RPS)], out_hbm.at[pl.ds(core * N + r0, RPS)])


BN = 2000


def _dense_body(act):
    def body(s_ref, nd_ref, b_ref, ns_ref, w_ref, o_ref):
        y = s_ref[0] * nd_ref[...] + b_ref[...]
        if act:
            y = jnp.where(y > 0, y, 0.01 * y)
        o_ref[0] = jnp.dot(y * ns_ref[...], w_ref[...],
                           preferred_element_type=jnp.float32)
    return body


def _dense_call(s, nd, b, ns, W, act):
    Din = s.shape[2]
    Dout = W.shape[1]
    return pl.pallas_call(
        _dense_body(act),
        grid=(B, N // BN),
        in_specs=[
            pl.BlockSpec((1, BN, Din), lambda c, i: (c, i, 0)),
            pl.BlockSpec((BN, 1), lambda c, i: (i, 0)),
            pl.BlockSpec((1, Din), lambda c, i: (0, 0)),
            pl.BlockSpec((BN, 1), lambda c, i: (i, 0)),
            pl.BlockSpec((Din, Dout), lambda c, i: (0, 0)),
        ],
        out_specs=pl.BlockSpec((1, BN, Dout), lambda c, i: (c, i, 0)),
        out_shape=jax.ShapeDtypeStruct((B, N, Dout), jnp.float32),
    )(s, nd, b, ns, W)


def _norm_body(deg_ref, ns_ref, nd_ref):
    ns_ref[...] = lax.rsqrt(deg_ref[0, :, 0:1])
    nd_ref[...] = lax.rsqrt(deg_ref[1, :, 0:1])


def _norm_call(deg):
    return pl.pallas_call(
        _norm_body,
        grid=(N // BN,),
        in_specs=[pl.BlockSpec((2, BN, 16), lambda i: (0, i, 0))],
        out_specs=[pl.BlockSpec((BN, 1), lambda i: (i, 0))] * 2,
        out_shape=[jax.ShapeDtypeStruct((N, 1), jnp.float32)] * 2,
    )(deg)


def _post3_body(s3_ref, h_ref, nd_ref, b3_ref, out_ref, hn_ref):
    o = s3_ref[0] * nd_ref[...] + b3_ref[...]
    out_ref[0] = o
    hn_ref[0] = jnp.concatenate([h_ref[0][:, 3:], o[:, :3]], axis=1)


def _post3_call(s3, h, nd, b3p):
    return pl.pallas_call(
        _post3_body,
        grid=(B, N // BN),
        in_specs=[
            pl.BlockSpec((1, BN, 16), lambda c, i: (c, i, 0)),
            pl.BlockSpec((1, BN, F), lambda c, i: (c, i, 0)),
            pl.BlockSpec((BN, 1), lambda c, i: (i, 0)),
            pl.BlockSpec((1, 16), lambda c, i: (0, 0)),
        ],
        out_specs=[
            pl.BlockSpec((1, BN, 16), lambda c, i: (c, i, 0)),
            pl.BlockSpec((1, BN, F), lambda c, i: (c, i, 0)),
        ],
        out_shape=[
            jax.ShapeDtypeStruct((B, N, 16), jnp.float32),
            jax.ShapeDtypeStruct((B, N, F), jnp.float32),
        ],
    )(s3, h, nd, b3p)


def kernel(edge_index, xx, output_length, W1, b1, W2, b2, W3, b3):
    src = edge_index[0].astype(jnp.int32)
    dst = edge_index[1].astype(jnp.int32)
    padn = E_PAD - E
    srcp = jnp.concatenate([src, jnp.zeros((padn,), jnp.int32)])
    dstp = jnp.concatenate([dst, jnp.zeros((padn,), jnp.int32)])

    init16 = jnp.tile(
        (lax.iota(jnp.int32, 16) == 0).astype(jnp.float32)[None, :], (N, 1))

    deg = _degrees(srcp, dstp, init16)
    ns, nd = _norm_call(deg.reshape(NCORE, N, 16))

    ones_n1 = jnp.ones((N, 1), jnp.float32)
    zeros_f = jnp.zeros((1, F), jnp.float32)
    b1r = b1.reshape(1, F)
    b2r = b2.reshape(1, F)
    W3p = jnp.pad(W3, ((0, 0), (0, 13)))
    b3p = jnp.pad(b3, (0, 13)).reshape(1, 16)

    h = xx
    outs = []
    for _ in range(2):
        z1 = _dense_call(h, ones_n1, zeros_f, ns, W1, act=False)
        s1 = _agg128(z1.reshape(NCORE * N, F), srcp, dstp)
        z2 = _dense_call(s1.reshape(B, N, F), nd, b1r, ns, W2, act=True)
        s2 = _agg128(z2.reshape(NCORE * N, F), srcp, dstp)
        p = _dense_call(s2.reshape(B, N, F), nd, b2r, ns, W3p, act=True)
        s3 = _agg16(p.reshape(NCORE * N, 16), srcp, dstp)
        out_t, h = _post3_call(s3.reshape(B, N, 16), h, nd, b3p)
        outs.append(out_t[:, :, :3])
    res = jnp.stack(outs, axis=2)
    res = res * (jnp.asarray(output_length) // 2).astype(res.dtype)
    return res


# speedup vs baseline: 65.4336x; 65.4336x over previous
import functools

import jax
import jax.numpy as jnp
from jax import lax
from jax.experimental import pallas as pl
from jax.experimental.pallas import tpu as pltpu
from jax.experimental.pallas import tpu_sc as plsc

N = 10000
F = 128
E = 320000
B = 2
NSUB = 16
NCORE = 2
CHUNK = 128
EPS = 20096
E_PAD = EPS * NSUB
NCHUNK = EPS // CHUNK
GARB = N
ACC_ROWS = N + 8
RPS_A = 632
RPS_LAST = N - 15 * RPS_A

_MESH = plsc.VectorSubcoreMesh(core_axis_name="c", subcore_axis_name="s")


def _rowcopy(sub, copy_fn):
    @pl.when(sub < NSUB - 1)
    def _():
        copy_fn(sub * RPS_A, RPS_A)

    @pl.when(sub == NSUB - 1)
    def _():
        copy_fn((NSUB - 1) * RPS_A, RPS_LAST)


def _make_agg(D):
    @functools.partial(
        pl.kernel,
        out_type=jax.ShapeDtypeStruct((NCORE * N, D), jnp.float32),
        mesh=_MESH,
        compiler_params=pltpu.CompilerParams(use_tc_tiling_on_sc=(D == F)),
        scratch_types=[
            pltpu.VMEM((CHUNK,), jnp.int32),
            pltpu.VMEM((CHUNK,), jnp.int32),
            pltpu.VMEM((CHUNK,), jnp.int32),
            pltpu.VMEM((CHUNK,), jnp.int32),
            pltpu.VMEM((CHUNK, D), jnp.float32),
            pltpu.VMEM_SHARED((ACC_ROWS, D), jnp.float32),
            pltpu.SemaphoreType.DMA,
        ],
    )
    def agg(z_hbm, src_hbm, dst_hbm, out_hbm, sbuf, dbuf, gbuf, wbuf, rows, acc, sem):
        core = lax.axis_index("c")
        sub = lax.axis_index("s")
        _rowcopy(sub, lambda r0, nr: pltpu.sync_copy(
            z_hbm.at[pl.ds(core * N + r0, nr)], acc.at[pl.ds(r0, nr)]))
        plsc.subcore_barrier()
        base = sub * EPS

        def body(ci, carry):
            off = base + ci * CHUNK
            pltpu.sync_copy(src_hbm.at[pl.ds(off, CHUNK)], sbuf)
            pltpu.sync_copy(dst_hbm.at[pl.ds(off, CHUNK)], dbuf)
            for j in range(CHUNK // 16):
                sl = pl.ds(j * 16, 16)
                s16 = sbuf[sl]
                d16 = dbuf[sl]
                gbuf[sl] = s16 + core * N
                wbuf[sl] = jnp.where(s16 == d16, GARB, d16)
            pltpu.async_copy(z_hbm.at[gbuf], rows, sem).wait()
            pltpu.sync_copy(rows, acc.at[wbuf], add=True)
            return carry

        lax.fori_loop(0, NCHUNK, body, 0)
        plsc.subcore_barrier()
        _rowcopy(sub, lambda r0, nr: pltpu.sync_copy(
            acc.at[pl.ds(r0, nr)], out_hbm.at[pl.ds(core * N + r0, nr)]))

    return agg


_agg128 = _make_agg(F)
_agg16 = _make_agg(16)


@functools.partial(
    pl.kernel,
    out_type=jax.ShapeDtypeStruct((NCORE * N, 16), jnp.float32),
    mesh=_MESH,
    compiler_params=pltpu.CompilerParams(use_tc_tiling_on_sc=False),
    scratch_types=[
        pltpu.VMEM((CHUNK,), jnp.int32),
        pltpu.VMEM((CHUNK,), jnp.int32),
        pltpu.VMEM((CHUNK,), jnp.int32),
        pltpu.VMEM((CHUNK, 16), jnp.float32),
        pltpu.VMEM_SHARED((ACC_ROWS, 16), jnp.float32),
    ],
)
def _degrees(src_hbm, dst_hbm, init_hbm, out_hbm, sbuf, dbuf, wbuf, ones, acc):
    core = lax.axis_index("c")
    sub = lax.axis_index("s")
    pltpu.sync_copy(init_hbm.at[pl.ds(0, CHUNK)], ones)
    _rowcopy(sub, lambda r0, nr: pltpu.sync_copy(
        init_hbm.at[pl.ds(r0, nr)], acc.at[pl.ds(r0, nr)]))
    plsc.subcore_barrier()
    base = sub * EPS

    def body(ci, carry):
        off = base + ci * CHUNK
        pltpu.sync_copy(src_hbm.at[pl.ds(off, CHUNK)], sbuf)
        pltpu.sync_copy(dst_hbm.at[pl.ds(off, CHUNK)], dbuf)
        for j in range(CHUNK // 16):
            sl = pl.ds(j * 16, 16)
            s16 = sbuf[sl]
            d16 = dbuf[sl]
            i16 = jnp.where(core == 0, s16, d16)
            wbuf[sl] = jnp.where(s16 == d16, GARB, i16)
        pltpu.sync_copy(ones, acc.at[wbuf], add=True)
        return carry

    lax.fori_loop(0, NCHUNK, body, 0)
    plsc.subcore_barrier()
    _rowcopy(sub, lambda r0, nr: pltpu.sync_copy(
        acc.at[pl.ds(r0, nr)], out_hbm.at[pl.ds(core * N + r0, nr)]))


BN = 2000


def _dense_body(act):
    def body(s_ref, nd_ref, b_ref, ns_ref, w_ref, o_ref):
        y = s_ref[0] * nd_ref[...] + b_ref[...]
        if act:
            y = jnp.where(y > 0, y, 0.01 * y)
        o_ref[0] = jnp.dot(y * ns_ref[...], w_ref[...],
                           preferred_element_type=jnp.float32)
    return body


def _dense_call(s, nd, b, ns, W, act):
    Din = s.shape[2]
    Dout = W.shape[1]
    return pl.pallas_call(
        _dense_body(act),
        grid=(B, N // BN),
        in_specs=[
            pl.BlockSpec((1, BN, Din), lambda c, i: (c, i, 0)),
            pl.BlockSpec((BN, 1), lambda c, i: (i, 0)),
            pl.BlockSpec((1, Din), lambda c, i: (0, 0)),
            pl.BlockSpec((BN, 1), lambda c, i: (i, 0)),
            pl.BlockSpec((Din, Dout), lambda c, i: (0, 0)),
        ],
        out_specs=pl.BlockSpec((1, BN, Dout), lambda c, i: (c, i, 0)),
        out_shape=jax.ShapeDtypeStruct((B, N, Dout), jnp.float32),
    )(s, nd, b, ns, W)


def _norm_body(deg_ref, ns_ref, nd_ref):
    ns_ref[...] = lax.rsqrt(deg_ref[0, :, 0:1])
    nd_ref[...] = lax.rsqrt(deg_ref[1, :, 0:1])


def _norm_call(deg):
    return pl.pallas_call(
        _norm_body,
        grid=(N // BN,),
        in_specs=[pl.BlockSpec((2, BN, 16), lambda i: (0, i, 0))],
        out_specs=[pl.BlockSpec((BN, 1), lambda i: (i, 0))] * 2,
        out_shape=[jax.ShapeDtypeStruct((N, 1), jnp.float32)] * 2,
    )(deg)


def _post3_body(s3_ref, h_ref, nd_ref, b3_ref, out_ref, hn_ref):
    o = s3_ref[0] * nd_ref[...] + b3_ref[...]
    out_ref[0] = o
    hn_ref[0] = jnp.concatenate([h_ref[0][:, 3:], o[:, :3]], axis=1)


def _post3_call(s3, h, nd, b3p):
    return pl.pallas_call(
        _post3_body,
        grid=(B, N // BN),
        in_specs=[
            pl.BlockSpec((1, BN, 16), lambda c, i: (c, i, 0)),
            pl.BlockSpec((1, BN, F), lambda c, i: (c, i, 0)),
            pl.BlockSpec((BN, 1), lambda c, i: (i, 0)),
            pl.BlockSpec((1, 16), lambda c, i: (0, 0)),
        ],
        out_specs=[
            pl.BlockSpec((1, BN, 16), lambda c, i: (c, i, 0)),
            pl.BlockSpec((1, BN, F), lambda c, i: (c, i, 0)),
        ],
        out_shape=[
            jax.ShapeDtypeStruct((B, N, 16), jnp.float32),
            jax.ShapeDtypeStruct((B, N, F), jnp.float32),
        ],
    )(s3, h, nd, b3p)


def kernel(edge_index, xx, output_length, W1, b1, W2, b2, W3, b3):
    src = edge_index[0].astype(jnp.int32)
    dst = edge_index[1].astype(jnp.int32)
    padn = E_PAD - E
    srcp = jnp.concatenate([src, jnp.zeros((padn,), jnp.int32)])
    dstp = jnp.concatenate([dst, jnp.zeros((padn,), jnp.int32)])

    init16 = jnp.tile(
        (lax.iota(jnp.int32, 16) == 0).astype(jnp.float32)[None, :], (N, 1))

    deg = _degrees(srcp, dstp, init16)
    ns, nd = _norm_call(deg.reshape(NCORE, N, 16))

    ones_n1 = jnp.ones((N, 1), jnp.float32)
    zeros_f = jnp.zeros((1, F), jnp.float32)
    b1r = b1.reshape(1, F)
    b2r = b2.reshape(1, F)
    W3p = jnp.pad(W3, ((0, 0), (0, 13)))
    b3p = jnp.pad(b3, (0, 13)).reshape(1, 16)

    h = xx
    outs = []
    for _ in range(2):
        z1 = _dense_call(h, ones_n1, zeros_f, ns, W1, act=False)
        s1 = _agg128(z1.reshape(NCORE * N, F), srcp, dstp)
        z2 = _dense_call(s1.reshape(B, N, F), nd, b1r, ns, W2, act=True)
        s2 = _agg128(z2.reshape(NCORE * N, F), srcp, dstp)
        p = _dense_call(s2.reshape(B, N, F), nd, b2r, ns, W3p, act=True)
        s3 = _agg16(p.reshape(NCORE * N, 16), srcp, dstp)
        out_t, h = _post3_call(s3.reshape(B, N, 16), h, nd, b3p)
        outs.append(out_t[:, :, :3])
    res = jnp.stack(outs, axis=2)
    res = res * (jnp.asarray(output_length) // 2).astype(res.dtype)
    return res


# speedup vs baseline: 78.0701x; 1.1931x over previous
import functools

import jax
import jax.numpy as jnp
from jax import lax
from jax.experimental import pallas as pl
from jax.experimental.pallas import tpu as pltpu
from jax.experimental.pallas import tpu_sc as plsc

N = 10000
F = 128
E = 320000
B = 2
NSUB = 16
NCORE = 2
CHUNK = 128
NCHUNK = 160
NK2 = NCHUNK // 2
EPS = NCHUNK * CHUNK
E_PAD = EPS * NSUB
GARB = N
ACC_ROWS = N + 8
RPS_A = 632
RPS_LAST = N - 15 * RPS_A

_MESH = plsc.VectorSubcoreMesh(core_axis_name="c", subcore_axis_name="s")


def _rowcopy(sub, copy_fn):
    @pl.when(sub < NSUB - 1)
    def _():
        copy_fn(sub * RPS_A, RPS_A)

    @pl.when(sub == NSUB - 1)
    def _():
        copy_fn((NSUB - 1) * RPS_A, RPS_LAST)


def _make_agg(D):
    @functools.partial(
        pl.kernel,
        out_type=jax.ShapeDtypeStruct((NCORE * N, D), jnp.float32),
        mesh=_MESH,
        compiler_params=pltpu.CompilerParams(use_tc_tiling_on_sc=(D == F)),
        scratch_types=[
            pltpu.VMEM((2, CHUNK), jnp.int32),
            pltpu.VMEM((2, CHUNK), jnp.int32),
            pltpu.VMEM((2, CHUNK), jnp.int32),
            pltpu.VMEM((2, CHUNK), jnp.int32),
            pltpu.VMEM((CHUNK, D), jnp.float32),
            pltpu.VMEM((CHUNK, D), jnp.float32),
            pltpu.VMEM_SHARED((ACC_ROWS, D), jnp.float32),
            pltpu.SemaphoreType.DMA,
            pltpu.SemaphoreType.DMA,
            pltpu.SemaphoreType.DMA,
            pltpu.SemaphoreType.DMA,
        ],
    )
    def agg(z_hbm, src_hbm, dst_hbm, out_hbm,
            sb, db, gb, wb, rows0, rows1, acc, semi0, semi1, semg0, semg1):
        core = lax.axis_index("c")
        sub = lax.axis_index("s")
        _rowcopy(sub, lambda r0, nr: pltpu.sync_copy(
            z_hbm.at[pl.ds(core * N + r0, nr)], acc.at[pl.ds(r0, nr)]))
        plsc.subcore_barrier()

        base = sub * EPS
        rows = (rows0, rows1)
        semi = (semi0, semi1)
        semg = (semg0, semg1)

        def idx_start(c, p):
            off = base + c * CHUNK
            pltpu.async_copy(src_hbm.at[pl.ds(off, CHUNK)], sb.at[p], semi[p])
            pltpu.async_copy(dst_hbm.at[pl.ds(off, CHUNK)], db.at[p], semi[p])

        def idx_wait(p):
            pltpu.make_async_copy(src_hbm.at[pl.ds(0, CHUNK)], sb.at[p], semi[p]).wait()
            pltpu.make_async_copy(dst_hbm.at[pl.ds(0, CHUNK)], db.at[p], semi[p]).wait()

        def transform(p):
            for j in range(CHUNK // 16):
                sl = pl.ds(j * 16, 16)
                s16 = sb[p, sl]
                d16 = db[p, sl]
                gb[p, sl] = s16 + core * N
                wb[p, sl] = jnp.where(s16 == d16, GARB, d16)

        def gather_start(p):
            pltpu.async_copy(z_hbm.at[gb.at[p]], rows[p], semg[p])

        def gather_wait(p):
            pltpu.make_async_copy(z_hbm.at[gb.at[p]], rows[p], semg[p]).wait()

        def scatter(p):
            pltpu.sync_copy(rows[p], acc.at[wb.at[p]], add=True)

        idx_start(0, 0)
        idx_start(1, 1)
        idx_wait(0)
        transform(0)
        gather_start(0)
        idx_start(2, 0)

        def body(k2, carry):
            idx_wait(1)
            transform(1)
            gather_start(1)

            @pl.when(k2 < NK2 - 1)
            def _():
                idx_start(2 * k2 + 3, 1)

            gather_wait(0)
            scatter(0)

            @pl.when(k2 < NK2 - 1)
            def _():
                idx_wait(0)
                transform(0)
                gather_start(0)

            @pl.when(k2 < NK2 - 2)
            def _():
                idx_start(2 * k2 + 4, 0)

            gather_wait(1)
            scatter(1)
            return carry

        lax.fori_loop(0, NK2, body, 0)
        plsc.subcore_barrier()
        _rowcopy(sub, lambda r0, nr: pltpu.sync_copy(
            acc.at[pl.ds(r0, nr)], out_hbm.at[pl.ds(core * N + r0, nr)]))

    return agg


_agg128 = _make_agg(F)
_agg16 = _make_agg(16)


@functools.partial(
    pl.kernel,
    out_type=jax.ShapeDtypeStruct((NCORE * N, 16), jnp.float32),
    mesh=_MESH,
    compiler_params=pltpu.CompilerParams(use_tc_tiling_on_sc=False),
    scratch_types=[
        pltpu.VMEM((2, CHUNK), jnp.int32),
        pltpu.VMEM((2, CHUNK), jnp.int32),
        pltpu.VMEM((2, CHUNK), jnp.int32),
        pltpu.VMEM((CHUNK, 16), jnp.float32),
        pltpu.VMEM_SHARED((ACC_ROWS, 16), jnp.float32),
        pltpu.SemaphoreType.DMA,
        pltpu.SemaphoreType.DMA,
    ],
)
def _degrees(src_hbm, dst_hbm, init_hbm, out_hbm,
             sb, db, wb, ones, acc, semi0, semi1):
    core = lax.axis_index("c")
    sub = lax.axis_index("s")
    pltpu.sync_copy(init_hbm.at[pl.ds(0, CHUNK)], ones)
    _rowcopy(sub, lambda r0, nr: pltpu.sync_copy(
        init_hbm.at[pl.ds(r0, nr)], acc.at[pl.ds(r0, nr)]))
    plsc.subcore_barrier()

    base = sub * EPS
    semi = (semi0, semi1)

    def idx_start(c, p):
        off = base + c * CHUNK
        pltpu.async_copy(src_hbm.at[pl.ds(off, CHUNK)], sb.at[p], semi[p])
        pltpu.async_copy(dst_hbm.at[pl.ds(off, CHUNK)], db.at[p], semi[p])

    def idx_wait(p):
        pltpu.make_async_copy(src_hbm.at[pl.ds(0, CHUNK)], sb.at[p], semi[p]).wait()
        pltpu.make_async_copy(dst_hbm.at[pl.ds(0, CHUNK)], db.at[p], semi[p]).wait()

    def step(p):
        for j in range(CHUNK // 16):
            sl = pl.ds(j * 16, 16)
            s16 = sb[p, sl]
            d16 = db[p, sl]
            i16 = jnp.where(core == 0, s16, d16)
            wb[p, sl] = jnp.where(s16 == d16, GARB, i16)
        pltpu.sync_copy(ones, acc.at[wb.at[p]], add=True)

    idx_start(0, 0)
    idx_start(1, 1)

    def body(k2, carry):
        idx_wait(0)

        @pl.when(k2 < NK2 - 1)
        def _():
            idx_start(2 * k2 + 2, 0)

        step(0)
        idx_wait(1)

        @pl.when(k2 < NK2 - 1)
        def _():
            idx_start(2 * k2 + 3, 1)

        step(1)
        return carry

    lax.fori_loop(0, NK2, body, 0)
    plsc.subcore_barrier()
    _rowcopy(sub, lambda r0, nr: pltpu.sync_copy(
        acc.at[pl.ds(r0, nr)], out_hbm.at[pl.ds(core * N + r0, nr)]))


BN = 2000


def _dense_body(act):
    def body(s_ref, nd_ref, b_ref, ns_ref, w_ref, o_ref):
        y = s_ref[0] * nd_ref[...] + b_ref[...]
        if act:
            y = jnp.where(y > 0, y, 0.01 * y)
        o_ref[0] = jnp.dot(y * ns_ref[...], w_ref[...],
                           preferred_element_type=jnp.float32)
    return body


def _dense_call(s, nd, b, ns, W, act):
    Din = s.shape[2]
    Dout = W.shape[1]
    return pl.pallas_call(
        _dense_body(act),
        grid=(B, N // BN),
        in_specs=[
            pl.BlockSpec((1, BN, Din), lambda c, i: (c, i, 0)),
            pl.BlockSpec((BN, 1), lambda c, i: (i, 0)),
            pl.BlockSpec((1, Din), lambda c, i: (0, 0)),
            pl.BlockSpec((BN, 1), lambda c, i: (i, 0)),
            pl.BlockSpec((Din, Dout), lambda c, i: (0, 0)),
        ],
        out_specs=pl.BlockSpec((1, BN, Dout), lambda c, i: (c, i, 0)),
        out_shape=jax.ShapeDtypeStruct((B, N, Dout), jnp.float32),
    )(s, nd, b, ns, W)


def _norm_body(deg_ref, ns_ref, nd_ref):
    ns_ref[...] = lax.rsqrt(deg_ref[0, :, 0:1])
    nd_ref[...] = lax.rsqrt(deg_ref[1, :, 0:1])


def _norm_call(deg):
    return pl.pallas_call(
        _norm_body,
        grid=(N // BN,),
        in_specs=[pl.BlockSpec((2, BN, 16), lambda i: (0, i, 0))],
        out_specs=[pl.BlockSpec((BN, 1), lambda i: (i, 0))] * 2,
        out_shape=[jax.ShapeDtypeStruct((N, 1), jnp.float32)] * 2,
    )(deg)


def _post3_body(s3_ref, h_ref, nd_ref, b3_ref, out_ref, hn_ref):
    o = s3_ref[0] * nd_ref[...] + b3_ref[...]
    out_ref[0] = o
    hn_ref[0] = jnp.concatenate([h_ref[0][:, 3:], o[:, :3]], axis=1)


def _post3_call(s3, h, nd, b3p):
    return pl.pallas_call(
        _post3_body,
        grid=(B, N // BN),
        in_specs=[
            pl.BlockSpec((1, BN, 16), lambda c, i: (c, i, 0)),
            pl.BlockSpec((1, BN, F), lambda c, i: (c, i, 0)),
            pl.BlockSpec((BN, 1), lambda c, i: (i, 0)),
            pl.BlockSpec((1, 16), lambda c, i: (0, 0)),
        ],
        out_specs=[
            pl.BlockSpec((1, BN, 16), lambda c, i: (c, i, 0)),
            pl.BlockSpec((1, BN, F), lambda c, i: (c, i, 0)),
        ],
        out_shape=[
            jax.ShapeDtypeStruct((B, N, 16), jnp.float32),
            jax.ShapeDtypeStruct((B, N, F), jnp.float32),
        ],
    )(s3, h, nd, b3p)


def kernel(edge_index, xx, output_length, W1, b1, W2, b2, W3, b3):
    src = edge_index[0].astype(jnp.int32)
    dst = edge_index[1].astype(jnp.int32)
    padn = E_PAD - E
    srcp = jnp.concatenate([src, jnp.zeros((padn,), jnp.int32)])
    dstp = jnp.concatenate([dst, jnp.zeros((padn,), jnp.int32)])

    init16 = jnp.tile(
        (lax.iota(jnp.int32, 16) == 0).astype(jnp.float32)[None, :], (N, 1))

    deg = _degrees(srcp, dstp, init16)
    ns, nd = _norm_call(deg.reshape(NCORE, N, 16))

    ones_n1 = jnp.ones((N, 1), jnp.float32)
    zeros_f = jnp.zeros((1, F), jnp.float32)
    b1r = b1.reshape(1, F)
    b2r = b2.reshape(1, F)
    W3p = jnp.pad(W3, ((0, 0), (0, 13)))
    b3p = jnp.pad(b3, (0, 13)).reshape(1, 16)

    h = xx
    outs = []
    for _ in range(2):
        z1 = _dense_call(h, ones_n1, zeros_f, ns, W1, act=False)
        s1 = _agg128(z1.reshape(NCORE * N, F), srcp, dstp)
        z2 = _dense_call(s1.reshape(B, N, F), nd, b1r, ns, W2, act=True)
        s2 = _agg128(z2.reshape(NCORE * N, F), srcp, dstp)
        p = _dense_call(s2.reshape(B, N, F), nd, b2r, ns, W3p, act=True)
        s3 = _agg16(p.reshape(NCORE * N, 16), srcp, dstp)
        out_t, h = _post3_call(s3.reshape(B, N, 16), h, nd, b3p)
        outs.append(out_t[:, :, :3])
    res = jnp.stack(outs, axis=2)
    res = res * (jnp.asarray(output_length) // 2).astype(res.dtype)
    return res


# speedup vs baseline: 80.0281x; 1.0251x over previous
import functools

import jax
import jax.numpy as jnp
from jax import lax
from jax.experimental import pallas as pl
from jax.experimental.pallas import tpu as pltpu
from jax.experimental.pallas import tpu_sc as plsc

N = 10000
F = 128
E = 320000
B = 2
NSUB = 16
NCORE = 2
CHUNK = 128
NCHUNK = 160
NK2 = NCHUNK // 2
EPS = NCHUNK * CHUNK
E_PAD = EPS * NSUB
GARB = N
ACC_ROWS = N + 8
RPS_A = 632
RPS_LAST = N - 15 * RPS_A

_MESH = plsc.VectorSubcoreMesh(core_axis_name="c", subcore_axis_name="s")


def _rowcopy(sub, copy_fn):
    @pl.when(sub < NSUB - 1)
    def _():
        copy_fn(sub * RPS_A, RPS_A)

    @pl.when(sub == NSUB - 1)
    def _():
        copy_fn((NSUB - 1) * RPS_A, RPS_LAST)


def _make_agg(D):
    @functools.partial(
        pl.kernel,
        out_type=jax.ShapeDtypeStruct((NCORE * N, D), jnp.float32),
        mesh=_MESH,
        compiler_params=pltpu.CompilerParams(use_tc_tiling_on_sc=(D == F)),
        scratch_types=[
            pltpu.VMEM((2, CHUNK), jnp.int32),
            pltpu.VMEM((2, CHUNK), jnp.int32),
            pltpu.VMEM((2, CHUNK), jnp.int32),
            pltpu.VMEM((2, CHUNK), jnp.int32),
            pltpu.VMEM((CHUNK, D), jnp.float32),
            pltpu.VMEM((CHUNK, D), jnp.float32),
            pltpu.VMEM_SHARED((ACC_ROWS, D), jnp.float32),
            pltpu.SemaphoreType.DMA,
            pltpu.SemaphoreType.DMA,
            pltpu.SemaphoreType.DMA,
            pltpu.SemaphoreType.DMA,
        ],
    )
    def agg(z_hbm, src_hbm, dst_hbm, out_hbm,
            sb, db, gb, wb, rows0, rows1, acc, semi0, semi1, semg0, semg1):
        core = lax.axis_index("c")
        sub = lax.axis_index("s")
        _rowcopy(sub, lambda r0, nr: pltpu.sync_copy(
            z_hbm.at[pl.ds(core * N + r0, nr)], acc.at[pl.ds(r0, nr)]))
        plsc.subcore_barrier()

        base = sub * EPS
        rows = (rows0, rows1)
        semi = (semi0, semi1)
        semg = (semg0, semg1)

        def idx_start(c, p):
            off = base + c * CHUNK
            pltpu.async_copy(src_hbm.at[pl.ds(off, CHUNK)], sb.at[p], semi[p])
            pltpu.async_copy(dst_hbm.at[pl.ds(off, CHUNK)], db.at[p], semi[p])

        def idx_wait(p):
            pltpu.make_async_copy(src_hbm.at[pl.ds(0, CHUNK)], sb.at[p], semi[p]).wait()
            pltpu.make_async_copy(dst_hbm.at[pl.ds(0, CHUNK)], db.at[p], semi[p]).wait()

        def transform(p):
            for j in range(CHUNK // 16):
                sl = pl.ds(j * 16, 16)
                s16 = sb[p, sl]
                d16 = db[p, sl]
                gb[p, sl] = s16 + core * N
                wb[p, sl] = jnp.where(s16 == d16, GARB, d16)

        def gather_start(p):
            pltpu.async_copy(z_hbm.at[gb.at[p]], rows[p], semg[p])

        def gather_wait(p):
            pltpu.make_async_copy(z_hbm.at[gb.at[p]], rows[p], semg[p]).wait()

        def scatter(p):
            pass

        idx_start(0, 0)
        idx_start(1, 1)
        idx_wait(0)
        transform(0)
        gather_start(0)
        idx_start(2, 0)

        def body(k2, carry):
            idx_wait(1)
            transform(1)
            gather_start(1)

            @pl.when(k2 < NK2 - 1)
            def _():
                idx_start(2 * k2 + 3, 1)

            gather_wait(0)
            scatter(0)

            @pl.when(k2 < NK2 - 1)
            def _():
                idx_wait(0)
                transform(0)
                gather_start(0)

            @pl.when(k2 < NK2 - 2)
            def _():
                idx_start(2 * k2 + 4, 0)

            gather_wait(1)
            scatter(1)
            return carry

        lax.fori_loop(0, NK2, body, 0)
        plsc.subcore_barrier()
        _rowcopy(sub, lambda r0, nr: pltpu.sync_copy(
            acc.at[pl.ds(r0, nr)], out_hbm.at[pl.ds(core * N + r0, nr)]))

    return agg


_agg128 = _make_agg(F)
_agg16 = _make_agg(16)


@functools.partial(
    pl.kernel,
    out_type=jax.ShapeDtypeStruct((NCORE * N, 16), jnp.float32),
    mesh=_MESH,
    compiler_params=pltpu.CompilerParams(use_tc_tiling_on_sc=False),
    scratch_types=[
        pltpu.VMEM((2, CHUNK), jnp.int32),
        pltpu.VMEM((2, CHUNK), jnp.int32),
        pltpu.VMEM((2, CHUNK), jnp.int32),
        pltpu.VMEM((CHUNK, 16), jnp.float32),
        pltpu.VMEM_SHARED((ACC_ROWS, 16), jnp.float32),
        pltpu.SemaphoreType.DMA,
        pltpu.SemaphoreType.DMA,
    ],
)
def _degrees(src_hbm, dst_hbm, init_hbm, out_hbm,
             sb, db, wb, ones, acc, semi0, semi1):
    core = lax.axis_index("c")
    sub = lax.axis_index("s")
    pltpu.sync_copy(init_hbm.at[pl.ds(0, CHUNK)], ones)
    _rowcopy(sub, lambda r0, nr: pltpu.sync_copy(
        init_hbm.at[pl.ds(r0, nr)], acc.at[pl.ds(r0, nr)]))
    plsc.subcore_barrier()

    base = sub * EPS
    semi = (semi0, semi1)

    def idx_start(c, p):
        off = base + c * CHUNK
        pltpu.async_copy(src_hbm.at[pl.ds(off, CHUNK)], sb.at[p], semi[p])
        pltpu.async_copy(dst_hbm.at[pl.ds(off, CHUNK)], db.at[p], semi[p])

    def idx_wait(p):
        pltpu.make_async_copy(src_hbm.at[pl.ds(0, CHUNK)], sb.at[p], semi[p]).wait()
        pltpu.make_async_copy(dst_hbm.at[pl.ds(0, CHUNK)], db.at[p], semi[p]).wait()

    def step(p):
        for j in range(CHUNK // 16):
            sl = pl.ds(j * 16, 16)
            s16 = sb[p, sl]
            d16 = db[p, sl]
            i16 = jnp.where(core == 0, s16, d16)
            wb[p, sl] = jnp.where(s16 == d16, GARB, i16)
        pltpu.sync_copy(ones, acc.at[wb.at[p]], add=True)

    idx_start(0, 0)
    idx_start(1, 1)

    def body(k2, carry):
        idx_wait(0)

        @pl.when(k2 < NK2 - 1)
        def _():
            idx_start(2 * k2 + 2, 0)

        step(0)
        idx_wait(1)

        @pl.when(k2 < NK2 - 1)
        def _():
            idx_start(2 * k2 + 3, 1)

        step(1)
        return carry

    lax.fori_loop(0, NK2, body, 0)
    plsc.subcore_barrier()
    _rowcopy(sub, lambda r0, nr: pltpu.sync_copy(
        acc.at[pl.ds(r0, nr)], out_hbm.at[pl.ds(core * N + r0, nr)]))


BN = 2000


def _dense_body(act):
    def body(s_ref, nd_ref, b_ref, ns_ref, w_ref, o_ref):
        y = s_ref[0] * nd_ref[...] + b_ref[...]
        if act:
            y = jnp.where(y > 0, y, 0.01 * y)
        o_ref[0] = jnp.dot(y * ns_ref[...], w_ref[...],
                           preferred_element_type=jnp.float32)
    return body


def _dense_call(s, nd, b, ns, W, act):
    Din = s.shape[2]
    Dout = W.shape[1]
    return pl.pallas_call(
        _dense_body(act),
        grid=(B, N // BN),
        in_specs=[
            pl.BlockSpec((1, BN, Din), lambda c, i: (c, i, 0)),
            pl.BlockSpec((BN, 1), lambda c, i: (i, 0)),
            pl.BlockSpec((1, Din), lambda c, i: (0, 0)),
            pl.BlockSpec((BN, 1), lambda c, i: (i, 0)),
            pl.BlockSpec((Din, Dout), lambda c, i: (0, 0)),
        ],
        out_specs=pl.BlockSpec((1, BN, Dout), lambda c, i: (c, i, 0)),
        out_shape=jax.ShapeDtypeStruct((B, N, Dout), jnp.float32),
    )(s, nd, b, ns, W)


def _norm_body(deg_ref, ns_ref, nd_ref):
    ns_ref[...] = lax.rsqrt(deg_ref[0, :, 0:1])
    nd_ref[...] = lax.rsqrt(deg_ref[1, :, 0:1])


def _norm_call(deg):
    return pl.pallas_call(
        _norm_body,
        grid=(N // BN,),
        in_specs=[pl.BlockSpec((2, BN, 16), lambda i: (0, i, 0))],
        out_specs=[pl.BlockSpec((BN, 1), lambda i: (i, 0))] * 2,
        out_shape=[jax.ShapeDtypeStruct((N, 1), jnp.float32)] * 2,
    )(deg)


def _post3_body(s3_ref, h_ref, nd_ref, b3_ref, out_ref, hn_ref):
    o = s3_ref[0] * nd_ref[...] + b3_ref[...]
    out_ref[0] = o
    hn_ref[0] = jnp.concatenate([h_ref[0][:, 3:], o[:, :3]], axis=1)


def _post3_call(s3, h, nd, b3p):
    return pl.pallas_call(
        _post3_body,
        grid=(B, N // BN),
        in_specs=[
            pl.BlockSpec((1, BN, 16), lambda c, i: (c, i, 0)),
            pl.BlockSpec((1, BN, F), lambda c, i: (c, i, 0)),
            pl.BlockSpec((BN, 1), lambda c, i: (i, 0)),
            pl.BlockSpec((1, 16), lambda c, i: (0, 0)),
        ],
        out_specs=[
            pl.BlockSpec((1, BN, 16), lambda c, i: (c, i, 0)),
            pl.BlockSpec((1, BN, F), lambda c, i: (c, i, 0)),
        ],
        out_shape=[
            jax.ShapeDtypeStruct((B, N, 16), jnp.float32),
            jax.ShapeDtypeStruct((B, N, F), jnp.float32),
        ],
    )(s3, h, nd, b3p)


def kernel(edge_index, xx, output_length, W1, b1, W2, b2, W3, b3):
    src = edge_index[0].astype(jnp.int32)
    dst = edge_index[1].astype(jnp.int32)
    padn = E_PAD - E
    srcp = jnp.concatenate([src, jnp.zeros((padn,), jnp.int32)])
    dstp = jnp.concatenate([dst, jnp.zeros((padn,), jnp.int32)])

    init16 = jnp.tile(
        (lax.iota(jnp.int32, 16) == 0).astype(jnp.float32)[None, :], (N, 1))

    deg = _degrees(srcp, dstp, init16)
    ns, nd = _norm_call(deg.reshape(NCORE, N, 16))

    ones_n1 = jnp.ones((N, 1), jnp.float32)
    zeros_f = jnp.zeros((1, F), jnp.float32)
    b1r = b1.reshape(1, F)
    b2r = b2.reshape(1, F)
    W3p = jnp.pad(W3, ((0, 0), (0, 13)))
    b3p = jnp.pad(b3, (0, 13)).reshape(1, 16)

    h = xx
    outs = []
    for _ in range(2):
        z1 = _dense_call(h, ones_n1, zeros_f, ns, W1, act=False)
        s1 = _agg128(z1.reshape(NCORE * N, F), srcp, dstp)
        z2 = _dense_call(s1.reshape(B, N, F), nd, b1r, ns, W2, act=True)
        s2 = _agg128(z2.reshape(NCORE * N, F), srcp, dstp)
        p = _dense_call(s2.reshape(B, N, F), nd, b2r, ns, W3p, act=True)
        s3 = _agg16(p.reshape(NCORE * N, 16), srcp, dstp)
        out_t, h = _post3_call(s3.reshape(B, N, 16), h, nd, b3p)
        outs.append(out_t[:, :, :3])
    res = jnp.stack(outs, axis=2)
    res = res * (jnp.asarray(output_length) // 2).astype(res.dtype)
    return res


# speedup vs baseline: 128.0048x; 1.5995x over previous
import functools

import jax
import jax.numpy as jnp
from jax import lax
from jax.experimental import pallas as pl
from jax.experimental.pallas import tpu as pltpu
from jax.experimental.pallas import tpu_sc as plsc

N = 10000
F = 128
E = 320000
B = 2
NSUB = 16
NCORE = 2
CHUNK = 128
NCHUNK = 160
NK2 = NCHUNK // 2
EPS = NCHUNK * CHUNK
E_PAD = EPS * NSUB
GARB = N
ACC_ROWS = N + 8
RPN = N // NSUB
NSTG = 5
ZS_ROWS = 15 * RPN + NSTG * CHUNK
RPS_A = 632
RPS_LAST = N - 15 * RPS_A

_MESH = plsc.VectorSubcoreMesh(core_axis_name="c", subcore_axis_name="s")


def _rowcopy(sub, copy_fn):
    @pl.when(sub < NSUB - 1)
    def _():
        copy_fn(sub * RPS_A, RPS_A)

    @pl.when(sub == NSUB - 1)
    def _():
        copy_fn((NSUB - 1) * RPS_A, RPS_LAST)


def _agg128_kernel():
    @functools.partial(
        pl.kernel,
        out_type=jax.ShapeDtypeStruct((2 * NCORE * N, 64), jnp.float32),
        mesh=_MESH,
        compiler_params=pltpu.CompilerParams(use_tc_tiling_on_sc=False),
        scratch_types=[
            pltpu.VMEM((2, CHUNK), jnp.int32),
            pltpu.VMEM((2, CHUNK), jnp.int32),
            pltpu.VMEM((2, CHUNK), jnp.int32),
            pltpu.VMEM((2, CHUNK), jnp.int32),
            pltpu.VMEM((CHUNK,), jnp.int32),
            pltpu.VMEM((CHUNK,), jnp.int32),
            pltpu.VMEM((CHUNK, 64), jnp.float32),
            pltpu.VMEM((CHUNK, 64), jnp.float32),
            pltpu.VMEM_SHARED((ACC_ROWS, 64), jnp.float32),
            pltpu.VMEM_SHARED((ZS_ROWS, 64), jnp.float32),
            pltpu.SemaphoreType.DMA,
            pltpu.SemaphoreType.DMA,
            pltpu.SemaphoreType.DMA,
            pltpu.SemaphoreType.DMA,
        ],
    )
    def agg(z64, src_hbm, dst_hbm, zero_hbm, out64,
            sb, db, gb, wb, stg, og, rows0, rows1, acc, z_s,
            semi0, semi1, semg0, semg1):
        core = lax.axis_index("c")
        sub = lax.axis_index("s")
        base = sub * EPS
        rows = (rows0, rows1)
        semi = (semi0, semi1)
        semg = (semg0, semg1)
        iota = lax.iota(jnp.int32, 16)

        def idx_start(c, p):
            off = base + c * CHUNK
            pltpu.async_copy(src_hbm.at[pl.ds(off, CHUNK)], sb.at[p], semi[p])
            pltpu.async_copy(dst_hbm.at[pl.ds(off, CHUNK)], db.at[p], semi[p])

        def idx_wait(p):
            pltpu.make_async_copy(src_hbm.at[pl.ds(0, CHUNK)], sb.at[p], semi[p]).wait()
            pltpu.make_async_copy(dst_hbm.at[pl.ds(0, CHUNK)], db.at[p], semi[p]).wait()

        def transform(p):
            for j in range(CHUNK // 16):
                sl = pl.ds(j * 16, 16)
                s16 = sb[p, sl]
                d16 = db[p, sl]
                gb[p, sl] = s16
                wb[p, sl] = jnp.where(s16 == d16, GARB, d16)

        def gather_start(p):
            pltpu.async_copy(z_s.at[gb.at[p]], rows[p], semg[p])

        def gather_wait(p):
            pltpu.make_async_copy(z_s.at[gb.at[p]], rows[p], semg[p]).wait()

        def scatter(p):
            pltpu.sync_copy(rows[p], acc.at[wb.at[p]], add=True)

        for h in range(2):
            hoff = 2 * core * N + h
            _rowcopy(sub, lambda r0, nr: pltpu.sync_copy(
                zero_hbm.at[pl.ds(0, nr)], acc.at[pl.ds(r0, nr)]))
            for k in range(NSTG):
                r0 = sub * RPN + k * CHUNK
                for j in range(CHUNK // 16):
                    m = jnp.minimum(r0 + j * 16 + iota, N - 1)
                    stg[pl.ds(j * 16, 16)] = 2 * m + hoff
                pltpu.sync_copy(z64.at[stg], rows0)
                pltpu.sync_copy(rows0, z_s.at[pl.ds(r0, CHUNK)])
            plsc.subcore_barrier()

            idx_start(0, 0)
            idx_start(1, 1)
            idx_wait(0)
            transform(0)
            gather_start(0)
            idx_start(2, 0)

            def body(k2, carry):
                idx_wait(1)
                transform(1)
                gather_start(1)

                @pl.when(k2 < NK2 - 1)
                def _():
                    idx_start(2 * k2 + 3, 1)

                gather_wait(0)
                scatter(0)

                @pl.when(k2 < NK2 - 1)
                def _():
                    idx_wait(0)
                    transform(0)
                    gather_start(0)

                @pl.when(k2 < NK2 - 2)
                def _():
                    idx_start(2 * k2 + 4, 0)

                gather_wait(1)
                scatter(1)
                return carry

            lax.fori_loop(0, NK2, body, 0)
            plsc.subcore_barrier()

            for k in range(NSTG):
                r0 = sub * RPN + k * CHUNK
                for j in range(CHUNK // 16):
                    m = jnp.minimum(r0 + j * 16 + iota, N - 1)
                    stg[pl.ds(j * 16, 16)] = m
                    og[pl.ds(j * 16, 16)] = 2 * m + hoff
                pltpu.sync_copy(acc.at[stg], rows0)
                pltpu.sync_copy(rows0, out64.at[og])
            plsc.subcore_barrier()

    return agg


_agg128 = _agg128_kernel()


@functools.partial(
    pl.kernel,
    out_type=jax.ShapeDtypeStruct((NCORE * N, 16), jnp.float32),
    mesh=_MESH,
    compiler_params=pltpu.CompilerParams(use_tc_tiling_on_sc=False),
    scratch_types=[
        pltpu.VMEM((2, CHUNK), jnp.int32),
        pltpu.VMEM((2, CHUNK), jnp.int32),
        pltpu.VMEM((2, CHUNK), jnp.int32),
        pltpu.VMEM((2, CHUNK), jnp.int32),
        pltpu.VMEM((CHUNK, 16), jnp.float32),
        pltpu.VMEM((CHUNK, 16), jnp.float32),
        pltpu.VMEM_SHARED((ACC_ROWS, 16), jnp.float32),
        pltpu.VMEM_SHARED((N, 16), jnp.float32),
        pltpu.SemaphoreType.DMA,
        pltpu.SemaphoreType.DMA,
        pltpu.SemaphoreType.DMA,
        pltpu.SemaphoreType.DMA,
    ],
)
def _agg16(z_hbm, src_hbm, dst_hbm, out_hbm,
           sb, db, gb, wb, rows0, rows1, acc, z_s,
           semi0, semi1, semg0, semg1):
    core = lax.axis_index("c")
    sub = lax.axis_index("s")
    _rowcopy(sub, lambda r0, nr: pltpu.sync_copy(
        z_hbm.at[pl.ds(core * N + r0, nr)], acc.at[pl.ds(r0, nr)]))
    _rowcopy(sub, lambda r0, nr: pltpu.sync_copy(
        z_hbm.at[pl.ds(core * N + r0, nr)], z_s.at[pl.ds(r0, nr)]))
    plsc.subcore_barrier()

    base = sub * EPS
    rows = (rows0, rows1)
    semi = (semi0, semi1)
    semg = (semg0, semg1)

    def idx_start(c, p):
        off = base + c * CHUNK
        pltpu.async_copy(src_hbm.at[pl.ds(off, CHUNK)], sb.at[p], semi[p])
        pltpu.async_copy(dst_hbm.at[pl.ds(off, CHUNK)], db.at[p], semi[p])

    def idx_wait(p):
        pltpu.make_async_copy(src_hbm.at[pl.ds(0, CHUNK)], sb.at[p], semi[p]).wait()
        pltpu.make_async_copy(dst_hbm.at[pl.ds(0, CHUNK)], db.at[p], semi[p]).wait()

    def transform(p):
        for j in range(CHUNK // 16):
            sl = pl.ds(j * 16, 16)
            s16 = sb[p, sl]
            d16 = db[p, sl]
            gb[p, sl] = s16
            wb[p, sl] = jnp.where(s16 == d16, GARB, d16)

    def gather_start(p):
        pltpu.async_copy(z_s.at[gb.at[p]], rows[p], semg[p])

    def gather_wait(p):
        pltpu.make_async_copy(z_s.at[gb.at[p]], rows[p], semg[p]).wait()

    def scatter(p):
        pltpu.sync_copy(rows[p], acc.at[wb.at[p]], add=True)

    idx_start(0, 0)
    idx_start(1, 1)
    idx_wait(0)
    transform(0)
    gather_start(0)
    idx_start(2, 0)

    def body(k2, carry):
        idx_wait(1)
        transform(1)
        gather_start(1)

        @pl.when(k2 < NK2 - 1)
        def _():
            idx_start(2 * k2 + 3, 1)

        gather_wait(0)
        scatter(0)

        @pl.when(k2 < NK2 - 1)
        def _():
            idx_wait(0)
            transform(0)
            gather_start(0)

        @pl.when(k2 < NK2 - 2)
        def _():
            idx_start(2 * k2 + 4, 0)

        gather_wait(1)
        scatter(1)
        return carry

    lax.fori_loop(0, NK2, body, 0)
    plsc.subcore_barrier()
    _rowcopy(sub, lambda r0, nr: pltpu.sync_copy(
        acc.at[pl.ds(r0, nr)], out_hbm.at[pl.ds(core * N + r0, nr)]))


@functools.partial(
    pl.kernel,
    out_type=jax.ShapeDtypeStruct((NCORE * N, 16), jnp.float32),
    mesh=_MESH,
    compiler_params=pltpu.CompilerParams(use_tc_tiling_on_sc=False),
    scratch_types=[
        pltpu.VMEM((2, CHUNK), jnp.int32),
        pltpu.VMEM((2, CHUNK), jnp.int32),
        pltpu.VMEM((2, CHUNK), jnp.int32),
        pltpu.VMEM((CHUNK, 16), jnp.float32),
        pltpu.VMEM_SHARED((ACC_ROWS, 16), jnp.float32),
        pltpu.SemaphoreType.DMA,
        pltpu.SemaphoreType.DMA,
    ],
)
def _degrees(src_hbm, dst_hbm, init_hbm, out_hbm,
             sb, db, wb, ones, acc, semi0, semi1):
    core = lax.axis_index("c")
    sub = lax.axis_index("s")
    pltpu.sync_copy(init_hbm.at[pl.ds(0, CHUNK)], ones)
    _rowcopy(sub, lambda r0, nr: pltpu.sync_copy(
        init_hbm.at[pl.ds(r0, nr)], acc.at[pl.ds(r0, nr)]))
    plsc.subcore_barrier()

    base = sub * EPS
    semi = (semi0, semi1)

    def idx_start(c, p):
        off = base + c * CHUNK
        pltpu.async_copy(src_hbm.at[pl.ds(off, CHUNK)], sb.at[p], semi[p])
        pltpu.async_copy(dst_hbm.at[pl.ds(off, CHUNK)], db.at[p], semi[p])

    def idx_wait(p):
        pltpu.make_async_copy(src_hbm.at[pl.ds(0, CHUNK)], sb.at[p], semi[p]).wait()
        pltpu.make_async_copy(dst_hbm.at[pl.ds(0, CHUNK)], db.at[p], semi[p]).wait()

    def step(p):
        for j in range(CHUNK // 16):
            sl = pl.ds(j * 16, 16)
            s16 = sb[p, sl]
            d16 = db[p, sl]
            i16 = jnp.where(core == 0, s16, d16)
            wb[p, sl] = jnp.where(s16 == d16, GARB, i16)
        pltpu.sync_copy(ones, acc.at[wb.at[p]], add=True)

    idx_start(0, 0)
    idx_start(1, 1)

    def body(k2, carry):
        idx_wait(0)

        @pl.when(k2 < NK2 - 1)
        def _():
            idx_start(2 * k2 + 2, 0)

        step(0)
        idx_wait(1)

        @pl.when(k2 < NK2 - 1)
        def _():
            idx_start(2 * k2 + 3, 1)

        step(1)
        return carry

    lax.fori_loop(0, NK2, body, 0)
    plsc.subcore_barrier()
    _rowcopy(sub, lambda r0, nr: pltpu.sync_copy(
        acc.at[pl.ds(r0, nr)], out_hbm.at[pl.ds(core * N + r0, nr)]))


BN = 2000


def _pre_body(h_ref, ns_ref, w_ref, o_ref):
    o_ref[0] = jnp.dot(h_ref[0] * ns_ref[...], w_ref[...],
                       preferred_element_type=jnp.float32)


def _pre_call(h, ns, W):
    return pl.pallas_call(
        _pre_body,
        grid=(B, N // BN),
        in_specs=[
            pl.BlockSpec((1, BN, F), lambda c, i: (c, i, 0)),
            pl.BlockSpec((BN, 1), lambda c, i: (i, 0)),
            pl.BlockSpec((F, F), lambda c, i: (0, 0)),
        ],
        out_specs=pl.BlockSpec((1, BN, F), lambda c, i: (c, i, 0)),
        out_shape=jax.ShapeDtypeStruct((B, N, F), jnp.float32),
    )(h, ns, W)


def _mid_body(s_ref, zs_ref, nd_ref, b_ref, ns_ref, w_ref, o_ref):
    y = (s_ref[0] + zs_ref[0]) * nd_ref[...] + b_ref[...]
    y = jnp.where(y > 0, y, 0.01 * y)
    o_ref[0] = jnp.dot(y * ns_ref[...], w_ref[...],
                       preferred_element_type=jnp.float32)


def _mid_call(s, zs, nd, b, ns, W):
    Dout = W.shape[1]
    return pl.pallas_call(
        _mid_body,
        grid=(B, N // BN),
        in_specs=[
            pl.BlockSpec((1, BN, F), lambda c, i: (c, i, 0)),
            pl.BlockSpec((1, BN, F), lambda c, i: (c, i, 0)),
            pl.BlockSpec((BN, 1), lambda c, i: (i, 0)),
            pl.BlockSpec((1, F), lambda c, i: (0, 0)),
            pl.BlockSpec((BN, 1), lambda c, i: (i, 0)),
            pl.BlockSpec((F, Dout), lambda c, i: (0, 0)),
        ],
        out_specs=pl.BlockSpec((1, BN, Dout), lambda c, i: (c, i, 0)),
        out_shape=jax.ShapeDtypeStruct((B, N, Dout), jnp.float32),
    )(s, zs, nd, b, ns, W)


def _norm_body(deg_ref, ns_ref, nd_ref):
    ns_ref[...] = lax.rsqrt(deg_ref[0, :, 0:1])
    nd_ref[...] = lax.rsqrt(deg_ref[1, :, 0:1])


def _norm_call(deg):
    return pl.pallas_call(
        _norm_body,
        grid=(N // BN,),
        in_specs=[pl.BlockSpec((2, BN, 16), lambda i: (0, i, 0))],
        out_specs=[pl.BlockSpec((BN, 1), lambda i: (i, 0))] * 2,
        out_shape=[jax.ShapeDtypeStruct((N, 1), jnp.float32)] * 2,
    )(deg)


def _post3_body(s3_ref, h_ref, nd_ref, b3_ref, out_ref, hn_ref):
    o = s3_ref[0] * nd_ref[...] + b3_ref[...]
    out_ref[0] = o
    hn_ref[0] = jnp.concatenate([h_ref[0][:, 3:], o[:, :3]], axis=1)


def _post3_call(s3, h, nd, b3p):
    return pl.pallas_call(
        _post3_body,
        grid=(B, N // BN),
        in_specs=[
            pl.BlockSpec((1, BN, 16), lambda c, i: (c, i, 0)),
            pl.BlockSpec((1, BN, F), lambda c, i: (c, i, 0)),
            pl.BlockSpec((BN, 1), lambda c, i: (i, 0)),
            pl.BlockSpec((1, 16), lambda c, i: (0, 0)),
        ],
        out_specs=[
            pl.BlockSpec((1, BN, 16), lambda c, i: (c, i, 0)),
            pl.BlockSpec((1, BN, F), lambda c, i: (c, i, 0)),
        ],
        out_shape=[
            jax.ShapeDtypeStruct((B, N, 16), jnp.float32),
            jax.ShapeDtypeStruct((B, N, F), jnp.float32),
        ],
    )(s3, h, nd, b3p)


def kernel(edge_index, xx, output_length, W1, b1, W2, b2, W3, b3):
    src = edge_index[0].astype(jnp.int32)
    dst = edge_index[1].astype(jnp.int32)
    padn = E_PAD - E
    srcp = jnp.concatenate([src, jnp.zeros((padn,), jnp.int32)])
    dstp = jnp.concatenate([dst, jnp.zeros((padn,), jnp.int32)])

    init16 = jnp.tile(
        (lax.iota(jnp.int32, 16) == 0).astype(jnp.float32)[None, :], (N, 1))
    zeros64 = jnp.zeros((RPS_A, 64), jnp.float32)

    deg = _degrees(srcp, dstp, init16)
    ns, nd = _norm_call(deg.reshape(NCORE, N, 16))

    b1r = b1.reshape(1, F)
    b2r = b2.reshape(1, F)
    W3p = jnp.pad(W3, ((0, 0), (0, 13)))
    b3p = jnp.pad(b3, (0, 13)).reshape(1, 16)

    def agg_full(z):
        z64 = z.reshape(2 * NCORE * N, 64)
        s64 = _agg128(z64, srcp, dstp, zeros64)
        return s64.reshape(B, N, F)

    h = xx
    outs = []
    for _ in range(2):
        z1 = _pre_call(h, ns, W1)
        s1 = agg_full(z1)
        z2 = _mid_call(s1, z1, nd, b1r, ns, W2)
        s2 = agg_full(z2)
        p = _mid_call(s2, z2, nd, b2r, ns, W3p)
        s3 = _agg16(p.reshape(NCORE * N, 16), srcp, dstp)
        out_t, h = _post3_call(s3.reshape(B, N, 16), h, nd, b3p)
        outs.append(out_t[:, :, :3])
    res = jnp.stack(outs, axis=2)
    res = res * (jnp.asarray(output_length) // 2).astype(res.dtype)
    return res
